# initial kernel scaffold (unmeasured)
import jax
import jax.numpy as jnp
from jax import lax
from jax.experimental import pallas as pl
from jax.experimental.pallas import tpu as pltpu


def kernel(
    x,
):
    def body(*refs):
        pass

    out_shape = jax.ShapeDtypeStruct(..., jnp.float32)
    return pl.pallas_call(body, out_shape=out_shape)(...)


# baseline (device time: 711038 ns/iter reference)
import jax
import jax.numpy as jnp
from jax import lax
from jax.experimental import pallas as pl
from jax.experimental.pallas import tpu as pltpu

N_DEV = 8
RS_HOPS = N_DEV - 1
N_HOPS = 2 * (N_DEV - 1)


def kernel(x):
    m, n = x.shape
    chunk = m // N_DEV

    def body(x_ref, out_ref, comm_ref, send_sems, recv_sems, credit_sem, copy_sem):
        my = lax.axis_index("i")
        left = jnp.mod(my - 1, N_DEV)
        right = jnp.mod(my + 1, N_DEV)

        barrier_sem = pltpu.get_barrier_semaphore()
        for nbr in (left, right):
            pl.semaphore_signal(
                barrier_sem, inc=1,
                device_id=(nbr,), device_id_type=pl.DeviceIdType.MESH,
            )
        pl.semaphore_wait(barrier_sem, 2)

        def copy_out(slot, cidx):
            cp = pltpu.make_async_copy(
                comm_ref.at[slot],
                out_ref.at[pl.ds(cidx * chunk, chunk), :],
                copy_sem,
            )
            cp.start()
            cp.wait()

        for h in range(N_HOPS):
            r_slot = h % 2
            if h == 0:
                src = x_ref.at[pl.ds(my * chunk, chunk), :]
            else:
                src = comm_ref.at[(h - 1) % 2]
            if h >= 2:
                pl.semaphore_wait(credit_sem, 1)
            rdma = pltpu.make_async_remote_copy(
                src_ref=src,
                dst_ref=comm_ref.at[r_slot],
                send_sem=send_sems.at[r_slot],
                recv_sem=recv_sems.at[r_slot],
                device_id=(right,),
                device_id_type=pl.DeviceIdType.MESH,
            )
            rdma.start()
            rdma.wait_send()
            if 1 <= h < N_HOPS - 1:
                pl.semaphore_signal(
                    credit_sem, inc=1,
                    device_id=(left,), device_id_type=pl.DeviceIdType.MESH,
                )
            rdma.wait_recv()
            if h < RS_HOPS:
                c = jnp.mod(my - h - 1, N_DEV)
                comm_ref[r_slot] = comm_ref[r_slot] + x_ref[pl.ds(c * chunk, chunk), :]
                if h == RS_HOPS - 1:
                    copy_out(r_slot, jnp.mod(my + 1, N_DEV))
            else:
                k = h - RS_HOPS
                copy_out(r_slot, jnp.mod(my - k, N_DEV))

    return pl.pallas_call(
        body,
        out_shape=jax.ShapeDtypeStruct((m, n), x.dtype),
        in_specs=[pl.BlockSpec(memory_space=pltpu.VMEM)],
        out_specs=pl.BlockSpec(memory_space=pl.ANY),
        scratch_shapes=[
            pltpu.VMEM((2, chunk, n), x.dtype),
            pltpu.SemaphoreType.DMA((2,)),
            pltpu.SemaphoreType.DMA((2,)),
            pltpu.SemaphoreType.REGULAR,
            pltpu.SemaphoreType.DMA,
        ],
        compiler_params=pltpu.CompilerParams(
            collective_id=0,
            vmem_limit_bytes=48 * 1024 * 1024,
        ),
    )(x)


# device time: 400531 ns/iter; 1.7752x vs baseline; 1.7752x over previous
import jax
import jax.numpy as jnp
from jax import lax
from jax.experimental import pallas as pl
from jax.experimental.pallas import tpu as pltpu

N_DEV = 8
RS_HOPS = N_DEV - 1
N_HOPS = 2 * (N_DEV - 1)


def _perm(p):
    return jnp.where(p < 4, p, 11 - p)


def kernel(x):
    m, n = x.shape
    chunk = m // N_DEV
    half = chunk // 2

    def body(x_ref, out_ref, comm_r, comm_l,
             send_r, recv_r, send_l, recv_l,
             credit_r, credit_l, copy_r, copy_l):
        my = lax.axis_index("i")
        r = _perm(my)
        right = _perm(jnp.mod(r + 1, N_DEV))
        left = _perm(jnp.mod(r - 1, N_DEV))

        barrier_sem = pltpu.get_barrier_semaphore()
        for nbr in (left, right):
            pl.semaphore_signal(
                barrier_sem, inc=1,
                device_id=(nbr,), device_id_type=pl.DeviceIdType.MESH,
            )
        pl.semaphore_wait(barrier_sem, 2)

        def top(ref, c, size=half):
            return ref.at[pl.ds(c * chunk, size), :]

        def bot(ref, c, size=half):
            return ref.at[pl.ds(c * chunk + half, size), :]

        def copy_out(src_ref, dst_ref, sem):
            cp = pltpu.make_async_copy(src_ref, dst_ref, sem)
            cp.start()
            cp.wait()

        for h in range(N_HOPS):
            slot = h % 2
            prev = (h - 1) % 2
            if h == 0:
                src_r = top(x_ref, r)
                src_l = bot(x_ref, r)
            else:
                src_r = comm_r.at[prev]
                src_l = comm_l.at[prev]
            if h >= 2:
                pl.semaphore_wait(credit_r, 1)
                pl.semaphore_wait(credit_l, 1)
            rdma_r = pltpu.make_async_remote_copy(
                src_ref=src_r,
                dst_ref=comm_r.at[slot],
                send_sem=send_r.at[slot],
                recv_sem=recv_r.at[slot],
                device_id=(right,),
                device_id_type=pl.DeviceIdType.MESH,
            )
            rdma_l = pltpu.make_async_remote_copy(
                src_ref=src_l,
                dst_ref=comm_l.at[slot],
                send_sem=send_l.at[slot],
                recv_sem=recv_l.at[slot],
                device_id=(left,),
                device_id_type=pl.DeviceIdType.MESH,
            )
            rdma_r.start()
            rdma_l.start()
            rdma_r.wait_send()
            rdma_l.wait_send()
            if 1 <= h < N_HOPS - 1:
                pl.semaphore_signal(
                    credit_r, inc=1,
                    device_id=(left,), device_id_type=pl.DeviceIdType.MESH,
                )
                pl.semaphore_signal(
                    credit_l, inc=1,
                    device_id=(right,), device_id_type=pl.DeviceIdType.MESH,
                )
            rdma_r.wait_recv()
            if h < RS_HOPS:
                c = jnp.mod(r - h - 1, N_DEV)
                comm_r[slot] = comm_r[slot] + x_ref[pl.ds(c * chunk, half), :]
            rdma_l.wait_recv()
            if h < RS_HOPS:
                c = jnp.mod(r + h + 1, N_DEV)
                comm_l[slot] = (
                    comm_l[slot] + x_ref[pl.ds(c * chunk + half, half), :]
                )
                if h == RS_HOPS - 1:
                    copy_out(comm_r.at[slot], top(out_ref, jnp.mod(r + 1, N_DEV)),
                             copy_r)
                    copy_out(comm_l.at[slot], bot(out_ref, jnp.mod(r - 1, N_DEV)),
                             copy_l)
            else:
                k = h - RS_HOPS
                copy_out(comm_r.at[slot], top(out_ref, jnp.mod(r - k, N_DEV)),
                         copy_r)
                copy_out(comm_l.at[slot], bot(out_ref, jnp.mod(r + k, N_DEV)),
                         copy_l)

    return pl.pallas_call(
        body,
        out_shape=jax.ShapeDtypeStruct((m, n), x.dtype),
        in_specs=[pl.BlockSpec(memory_space=pltpu.VMEM)],
        out_specs=pl.BlockSpec(memory_space=pl.ANY),
        scratch_shapes=[
            pltpu.VMEM((2, half, n), x.dtype),
            pltpu.VMEM((2, half, n), x.dtype),
            pltpu.SemaphoreType.DMA((2,)),
            pltpu.SemaphoreType.DMA((2,)),
            pltpu.SemaphoreType.DMA((2,)),
            pltpu.SemaphoreType.DMA((2,)),
            pltpu.SemaphoreType.REGULAR,
            pltpu.SemaphoreType.REGULAR,
            pltpu.SemaphoreType.DMA,
            pltpu.SemaphoreType.DMA,
        ],
        compiler_params=pltpu.CompilerParams(
            collective_id=0,
            vmem_limit_bytes=48 * 1024 * 1024,
        ),
    )(x)


# device time: 381269 ns/iter; 1.8649x vs baseline; 1.0505x over previous
import jax
import jax.numpy as jnp
from jax import lax
from jax.experimental import pallas as pl
from jax.experimental.pallas import tpu as pltpu

N_DEV = 8
RS_HOPS = N_DEV - 1
N_HOPS = 2 * (N_DEV - 1)


def _perm(p):
    return jnp.where(p < 4, p, 11 - p)


def kernel(x):
    m, n = x.shape
    chunk = m // N_DEV
    half = chunk // 2

    def body(x_ref, out_ref, comm_r, comm_l,
             send_r, recv_r, send_l, recv_l,
             credit_r, credit_l, copy_r, copy_l):
        my = lax.axis_index("i")
        r = _perm(my)
        right = _perm(jnp.mod(r + 1, N_DEV))
        left = _perm(jnp.mod(r - 1, N_DEV))

        barrier_sem = pltpu.get_barrier_semaphore()
        for nbr in (left, right):
            pl.semaphore_signal(
                barrier_sem, inc=1,
                device_id=(nbr,), device_id_type=pl.DeviceIdType.MESH,
            )
        pl.semaphore_wait(barrier_sem, 2)

        def top(ref, c, size=half):
            return ref.at[pl.ds(c * chunk, size), :]

        def bot(ref, c, size=half):
            return ref.at[pl.ds(c * chunk + half, size), :]

        pending = []

        def copy_out_start(src_ref, dst_ref, sem):
            cp = pltpu.make_async_copy(src_ref, dst_ref, sem)
            cp.start()
            pending.append(cp)

        def drain_pending():
            while pending:
                pending.pop().wait()

        for h in range(N_HOPS):
            slot = h % 2
            prev = (h - 1) % 2
            if h == 0:
                src_r = top(x_ref, r)
                src_l = bot(x_ref, r)
            else:
                src_r = comm_r.at[prev]
                src_l = comm_l.at[prev]
            if h >= 2:
                pl.semaphore_wait(credit_r, 1)
                pl.semaphore_wait(credit_l, 1)
            rdma_r = pltpu.make_async_remote_copy(
                src_ref=src_r,
                dst_ref=comm_r.at[slot],
                send_sem=send_r.at[slot],
                recv_sem=recv_r.at[slot],
                device_id=(right,),
                device_id_type=pl.DeviceIdType.MESH,
            )
            rdma_l = pltpu.make_async_remote_copy(
                src_ref=src_l,
                dst_ref=comm_l.at[slot],
                send_sem=send_l.at[slot],
                recv_sem=recv_l.at[slot],
                device_id=(left,),
                device_id_type=pl.DeviceIdType.MESH,
            )
            rdma_r.start()
            rdma_l.start()
            rdma_r.wait_send()
            rdma_l.wait_send()
            drain_pending()
            if 1 <= h < N_HOPS - 1:
                pl.semaphore_signal(
                    credit_r, inc=1,
                    device_id=(left,), device_id_type=pl.DeviceIdType.MESH,
                )
                pl.semaphore_signal(
                    credit_l, inc=1,
                    device_id=(right,), device_id_type=pl.DeviceIdType.MESH,
                )
            rdma_r.wait_recv()
            if h < RS_HOPS:
                c = jnp.mod(r - h - 1, N_DEV)
                comm_r[slot] = comm_r[slot] + x_ref[pl.ds(c * chunk, half), :]
            rdma_l.wait_recv()
            if h < RS_HOPS:
                c = jnp.mod(r + h + 1, N_DEV)
                comm_l[slot] = (
                    comm_l[slot] + x_ref[pl.ds(c * chunk + half, half), :]
                )
                if h == RS_HOPS - 1:
                    copy_out_start(comm_r.at[slot],
                                   top(out_ref, jnp.mod(r + 1, N_DEV)), copy_r)
                    copy_out_start(comm_l.at[slot],
                                   bot(out_ref, jnp.mod(r - 1, N_DEV)), copy_l)
            else:
                k = h - RS_HOPS
                copy_out_start(comm_r.at[slot],
                               top(out_ref, jnp.mod(r - k, N_DEV)), copy_r)
                copy_out_start(comm_l.at[slot],
                               bot(out_ref, jnp.mod(r + k, N_DEV)), copy_l)
        drain_pending()

    return pl.pallas_call(
        body,
        out_shape=jax.ShapeDtypeStruct((m, n), x.dtype),
        in_specs=[pl.BlockSpec(memory_space=pltpu.VMEM)],
        out_specs=pl.BlockSpec(memory_space=pl.ANY),
        scratch_shapes=[
            pltpu.VMEM((2, half, n), x.dtype),
            pltpu.VMEM((2, half, n), x.dtype),
            pltpu.SemaphoreType.DMA((2,)),
            pltpu.SemaphoreType.DMA((2,)),
            pltpu.SemaphoreType.DMA((2,)),
            pltpu.SemaphoreType.DMA((2,)),
            pltpu.SemaphoreType.REGULAR,
            pltpu.SemaphoreType.REGULAR,
            pltpu.SemaphoreType.DMA,
            pltpu.SemaphoreType.DMA,
        ],
        compiler_params=pltpu.CompilerParams(
            collective_id=0,
            vmem_limit_bytes=48 * 1024 * 1024,
        ),
    )(x)


# device time: 379308 ns/iter; 1.8746x vs baseline; 1.0052x over previous
import jax
import jax.numpy as jnp
from jax import lax
from jax.experimental import pallas as pl
from jax.experimental.pallas import tpu as pltpu

N_DEV = 8
RS_HOPS = N_DEV - 1
N_HOPS = 2 * (N_DEV - 1)
N_SUB = 2


def _perm(p):
    return jnp.where(p < 4, p, 11 - p)


def kernel(x):
    m, n = x.shape
    chunk = m // N_DEV
    half = chunk // 2
    quart = half // N_SUB

    def body(x_ref, out_ref, comm_r, comm_l,
             send_r, recv_r, send_l, recv_l,
             credit_r, credit_l, copy_r, copy_l):
        my = lax.axis_index("i")
        r = _perm(my)
        right = _perm(jnp.mod(r + 1, N_DEV))
        left = _perm(jnp.mod(r - 1, N_DEV))

        barrier_sem = pltpu.get_barrier_semaphore()
        for nbr in (left, right):
            pl.semaphore_signal(
                barrier_sem, inc=1,
                device_id=(nbr,), device_id_type=pl.DeviceIdType.MESH,
            )
        pl.semaphore_wait(barrier_sem, 2)

        def top(ref, c, q):
            return ref.at[pl.ds(c * chunk + q * quart, quart), :]

        def bot(ref, c, q):
            return ref.at[pl.ds(c * chunk + half + q * quart, quart), :]

        pending = []

        def copy_out_start(src_ref, dst_ref, sem):
            cp = pltpu.make_async_copy(src_ref, dst_ref, sem)
            cp.start()
            pending.append(cp)

        def drain_pending():
            while pending:
                pending.pop().wait()

        for h in range(N_HOPS):
            slot = h % 2
            prev = (h - 1) % 2
            rdmas = []
            for q in range(N_SUB):
                if h == 0:
                    src_r = top(x_ref, r, q)
                    src_l = bot(x_ref, r, q)
                else:
                    src_r = comm_r.at[prev, q]
                    src_l = comm_l.at[prev, q]
                if q == 0 and h >= 2:
                    pl.semaphore_wait(credit_r, 1)
                    pl.semaphore_wait(credit_l, 1)
                rdma_r = pltpu.make_async_remote_copy(
                    src_ref=src_r,
                    dst_ref=comm_r.at[slot, q],
                    send_sem=send_r.at[slot, q],
                    recv_sem=recv_r.at[slot, q],
                    device_id=(right,),
                    device_id_type=pl.DeviceIdType.MESH,
                )
                rdma_l = pltpu.make_async_remote_copy(
                    src_ref=src_l,
                    dst_ref=comm_l.at[slot, q],
                    send_sem=send_l.at[slot, q],
                    recv_sem=recv_l.at[slot, q],
                    device_id=(left,),
                    device_id_type=pl.DeviceIdType.MESH,
                )
                rdma_r.start()
                rdma_l.start()
                rdmas.append((rdma_r, rdma_l))
            for q, (rdma_r, rdma_l) in enumerate(rdmas):
                rdma_r.wait_send()
                rdma_l.wait_send()
                if q == N_SUB - 1:
                    drain_pending()
                    if 1 <= h < N_HOPS - 1:
                        pl.semaphore_signal(
                            credit_r, inc=1,
                            device_id=(left,),
                            device_id_type=pl.DeviceIdType.MESH,
                        )
                        pl.semaphore_signal(
                            credit_l, inc=1,
                            device_id=(right,),
                            device_id_type=pl.DeviceIdType.MESH,
                        )
                rdma_r.wait_recv()
                rdma_l.wait_recv()
                if h < RS_HOPS:
                    cr = jnp.mod(r - h - 1, N_DEV)
                    cl = jnp.mod(r + h + 1, N_DEV)
                    comm_r[slot, q] = (
                        comm_r[slot, q]
                        + x_ref[pl.ds(cr * chunk + q * quart, quart), :]
                    )
                    comm_l[slot, q] = (
                        comm_l[slot, q]
                        + x_ref[pl.ds(cl * chunk + half + q * quart, quart), :]
                    )
                    if h == RS_HOPS - 1:
                        copy_out_start(
                            comm_r.at[slot, q],
                            top(out_ref, jnp.mod(r + 1, N_DEV), q),
                            copy_r.at[q],
                        )
                        copy_out_start(
                            comm_l.at[slot, q],
                            bot(out_ref, jnp.mod(r - 1, N_DEV), q),
                            copy_l.at[q],
                        )
                else:
                    k = h - RS_HOPS
                    copy_out_start(
                        comm_r.at[slot, q],
                        top(out_ref, jnp.mod(r - k, N_DEV), q),
                        copy_r.at[q],
                    )
                    copy_out_start(
                        comm_l.at[slot, q],
                        bot(out_ref, jnp.mod(r + k, N_DEV), q),
                        copy_l.at[q],
                    )
        drain_pending()

    return pl.pallas_call(
        body,
        out_shape=jax.ShapeDtypeStruct((m, n), x.dtype),
        in_specs=[pl.BlockSpec(memory_space=pltpu.VMEM)],
        out_specs=pl.BlockSpec(memory_space=pl.ANY),
        scratch_shapes=[
            pltpu.VMEM((2, N_SUB, quart, n), x.dtype),
            pltpu.VMEM((2, N_SUB, quart, n), x.dtype),
            pltpu.SemaphoreType.DMA((2, N_SUB)),
            pltpu.SemaphoreType.DMA((2, N_SUB)),
            pltpu.SemaphoreType.DMA((2, N_SUB)),
            pltpu.SemaphoreType.DMA((2, N_SUB)),
            pltpu.SemaphoreType.REGULAR,
            pltpu.SemaphoreType.REGULAR,
            pltpu.SemaphoreType.DMA((N_SUB,)),
            pltpu.SemaphoreType.DMA((N_SUB,)),
        ],
        compiler_params=pltpu.CompilerParams(
            collective_id=0,
            vmem_limit_bytes=48 * 1024 * 1024,
        ),
    )(x)


# device time: 356434 ns/iter; 1.9949x vs baseline; 1.0642x over previous
import jax
import jax.numpy as jnp
from jax import lax
from jax.experimental import pallas as pl
from jax.experimental.pallas import tpu as pltpu

N_DEV = 8
RS_HOPS = N_DEV - 1
N_HOPS = 2 * (N_DEV - 1)
N_SUB = 2


def _perm(p):
    return jnp.where(p < 4, p, 11 - p)


def kernel(x):
    m, n = x.shape
    chunk = m // N_DEV
    half = chunk // 2
    quart = half // N_SUB

    def body(x_ref, out_ref, comm_r, comm_l,
             send_r, recv_r, send_l, recv_l,
             credit_r, credit_l, copy_r, copy_l):
        my = lax.axis_index("i")
        r = _perm(my)
        right = _perm(jnp.mod(r + 1, N_DEV))
        left = _perm(jnp.mod(r - 1, N_DEV))

        barrier_sem = pltpu.get_barrier_semaphore()
        for nbr in (left, right):
            pl.semaphore_signal(
                barrier_sem, inc=1,
                device_id=(nbr,), device_id_type=pl.DeviceIdType.MESH,
            )
        pl.semaphore_wait(barrier_sem, 2)

        def top(ref, c, q):
            return ref.at[pl.ds(c * chunk + q * quart, quart), :]

        def bot(ref, c, q):
            return ref.at[pl.ds(c * chunk + half + q * quart, quart), :]

        streams = []
        for q in range(N_SUB):
            streams.append(dict(
                comm=comm_r, send=send_r, recv=recv_r, credit=credit_r,
                copy=copy_r, slc=top, dst=right, ups=left, sgn=+1, q=q,
            ))
            streams.append(dict(
                comm=comm_l, send=send_l, recv=recv_l, credit=credit_l,
                copy=copy_l, slc=bot, dst=left, ups=right, sgn=-1, q=q,
            ))

        def make_rdma(s, h):
            slot = h % 2
            q = s["q"]
            if h == 0:
                src = s["slc"](x_ref, r, q)
            else:
                src = s["comm"].at[(h - 1) % 2, q]
            return pltpu.make_async_remote_copy(
                src_ref=src,
                dst_ref=s["comm"].at[slot, q],
                send_sem=s["send"].at[slot, q],
                recv_sem=s["recv"].at[slot, q],
                device_id=(s["dst"],),
                device_id_type=pl.DeviceIdType.MESH,
            )

        def start_hop(s, h):
            if h >= 2:
                pl.semaphore_wait(s["credit"].at[s["q"]], 1)
            rdma = make_rdma(s, h)
            rdma.start()
            s["rdma"] = rdma

        for s in streams:
            start_hop(s, 0)

        for h in range(N_HOPS):
            slot = h % 2
            for s in streams:
                q = s["q"]
                rdma = s["rdma"]
                rdma.wait_send()
                cp = s.pop("cp", None)
                if cp is not None:
                    cp.wait()
                if 1 <= h < N_HOPS - 1:
                    pl.semaphore_signal(
                        s["credit"].at[q], inc=1,
                        device_id=(s["ups"],),
                        device_id_type=pl.DeviceIdType.MESH,
                    )
                rdma.wait_recv()
                if h < RS_HOPS:
                    c = jnp.mod(r - s["sgn"] * (h + 1), N_DEV)
                    s["comm"][slot, q] = (
                        s["comm"][slot, q] + s["slc"](x_ref, c, q)[...]
                    )
                    if h == RS_HOPS - 1:
                        own = jnp.mod(r + s["sgn"], N_DEV)
                        cp = pltpu.make_async_copy(
                            s["comm"].at[slot, q],
                            s["slc"](out_ref, own, q),
                            s["copy"].at[q],
                        )
                        cp.start()
                        s["cp"] = cp
                else:
                    k = h - RS_HOPS
                    c = jnp.mod(r - s["sgn"] * k, N_DEV)
                    cp = pltpu.make_async_copy(
                        s["comm"].at[slot, q],
                        s["slc"](out_ref, c, q),
                        s["copy"].at[q],
                    )
                    cp.start()
                    s["cp"] = cp
                if h + 1 < N_HOPS:
                    start_hop(s, h + 1)

        for s in streams:
            cp = s.pop("cp", None)
            if cp is not None:
                cp.wait()

    return pl.pallas_call(
        body,
        out_shape=jax.ShapeDtypeStruct((m, n), x.dtype),
        in_specs=[pl.BlockSpec(memory_space=pltpu.VMEM)],
        out_specs=pl.BlockSpec(memory_space=pl.ANY),
        scratch_shapes=[
            pltpu.VMEM((2, N_SUB, quart, n), x.dtype),
            pltpu.VMEM((2, N_SUB, quart, n), x.dtype),
            pltpu.SemaphoreType.DMA((2, N_SUB)),
            pltpu.SemaphoreType.DMA((2, N_SUB)),
            pltpu.SemaphoreType.DMA((2, N_SUB)),
            pltpu.SemaphoreType.DMA((2, N_SUB)),
            pltpu.SemaphoreType.REGULAR((N_SUB,)),
            pltpu.SemaphoreType.REGULAR((N_SUB,)),
            pltpu.SemaphoreType.DMA((N_SUB,)),
            pltpu.SemaphoreType.DMA((N_SUB,)),
        ],
        compiler_params=pltpu.CompilerParams(
            collective_id=0,
            vmem_limit_bytes=48 * 1024 * 1024,
        ),
    )(x)


# device time: 355831 ns/iter; 1.9982x vs baseline; 1.0017x over previous
import jax
import jax.numpy as jnp
from jax import lax
from jax.experimental import pallas as pl
from jax.experimental.pallas import tpu as pltpu

N_DEV = 8
RS_HOPS = N_DEV - 1
N_HOPS = 2 * (N_DEV - 1)
N_SUB = 4


def _perm(p):
    return jnp.where(p < 4, p, 11 - p)


def kernel(x):
    m, n = x.shape
    chunk = m // N_DEV
    half = chunk // 2
    quart = half // N_SUB

    def body(x_ref, out_ref, comm_r, comm_l,
             send_r, recv_r, send_l, recv_l,
             credit_r, credit_l, copy_r, copy_l):
        my = lax.axis_index("i")
        r = _perm(my)
        right = _perm(jnp.mod(r + 1, N_DEV))
        left = _perm(jnp.mod(r - 1, N_DEV))

        barrier_sem = pltpu.get_barrier_semaphore()
        for nbr in (left, right):
            pl.semaphore_signal(
                barrier_sem, inc=1,
                device_id=(nbr,), device_id_type=pl.DeviceIdType.MESH,
            )
        pl.semaphore_wait(barrier_sem, 2)

        def top(ref, c, q):
            return ref.at[pl.ds(c * chunk + q * quart, quart), :]

        def bot(ref, c, q):
            return ref.at[pl.ds(c * chunk + half + q * quart, quart), :]

        streams = []
        for q in range(N_SUB):
            streams.append(dict(
                comm=comm_r, send=send_r, recv=recv_r, credit=credit_r,
                copy=copy_r, slc=top, dst=right, ups=left, sgn=+1, q=q,
            ))
            streams.append(dict(
                comm=comm_l, send=send_l, recv=recv_l, credit=credit_l,
                copy=copy_l, slc=bot, dst=left, ups=right, sgn=-1, q=q,
            ))

        def make_rdma(s, h):
            slot = h % 2
            q = s["q"]
            if h == 0:
                src = s["slc"](x_ref, r, q)
            else:
                src = s["comm"].at[(h - 1) % 2, q]
            return pltpu.make_async_remote_copy(
                src_ref=src,
                dst_ref=s["comm"].at[slot, q],
                send_sem=s["send"].at[slot, q],
                recv_sem=s["recv"].at[slot, q],
                device_id=(s["dst"],),
                device_id_type=pl.DeviceIdType.MESH,
            )

        def start_hop(s, h):
            if h >= 2:
                pl.semaphore_wait(s["credit"].at[s["q"]], 1)
            rdma = make_rdma(s, h)
            rdma.start()
            s["rdma"] = rdma

        for s in streams:
            start_hop(s, 0)

        for h in range(N_HOPS):
            slot = h % 2
            for s in streams:
                q = s["q"]
                rdma = s["rdma"]
                rdma.wait_send()
                cp = s.pop("cp", None)
                if cp is not None:
                    cp.wait()
                if 1 <= h < N_HOPS - 1:
                    pl.semaphore_signal(
                        s["credit"].at[q], inc=1,
                        device_id=(s["ups"],),
                        device_id_type=pl.DeviceIdType.MESH,
                    )
                rdma.wait_recv()
                if h < RS_HOPS:
                    c = jnp.mod(r - s["sgn"] * (h + 1), N_DEV)
                    s["comm"][slot, q] = (
                        s["comm"][slot, q] + s["slc"](x_ref, c, q)[...]
                    )
                    if h == RS_HOPS - 1:
                        own = jnp.mod(r + s["sgn"], N_DEV)
                        cp = pltpu.make_async_copy(
                            s["comm"].at[slot, q],
                            s["slc"](out_ref, own, q),
                            s["copy"].at[q],
                        )
                        cp.start()
                        s["cp"] = cp
                else:
                    k = h - RS_HOPS
                    c = jnp.mod(r - s["sgn"] * k, N_DEV)
                    cp = pltpu.make_async_copy(
                        s["comm"].at[slot, q],
                        s["slc"](out_ref, c, q),
                        s["copy"].at[q],
                    )
                    cp.start()
                    s["cp"] = cp
                if h + 1 < N_HOPS:
                    start_hop(s, h + 1)

        for s in streams:
            cp = s.pop("cp", None)
            if cp is not None:
                cp.wait()

    return pl.pallas_call(
        body,
        out_shape=jax.ShapeDtypeStruct((m, n), x.dtype),
        in_specs=[pl.BlockSpec(memory_space=pltpu.VMEM)],
        out_specs=pl.BlockSpec(memory_space=pl.ANY),
        scratch_shapes=[
            pltpu.VMEM((2, N_SUB, quart, n), x.dtype),
            pltpu.VMEM((2, N_SUB, quart, n), x.dtype),
            pltpu.SemaphoreType.DMA((2, N_SUB)),
            pltpu.SemaphoreType.DMA((2, N_SUB)),
            pltpu.SemaphoreType.DMA((2, N_SUB)),
            pltpu.SemaphoreType.DMA((2, N_SUB)),
            pltpu.SemaphoreType.REGULAR((N_SUB,)),
            pltpu.SemaphoreType.REGULAR((N_SUB,)),
            pltpu.SemaphoreType.DMA((N_SUB,)),
            pltpu.SemaphoreType.DMA((N_SUB,)),
        ],
        compiler_params=pltpu.CompilerParams(
            collective_id=0,
            vmem_limit_bytes=48 * 1024 * 1024,
        ),
    )(x)
